# baseline (speedup 1.0000x reference)
import functools

import jax
import jax.numpy as jnp
from jax import lax
from jax.experimental import pallas as pl
from jax.experimental.pallas import tpu as pltpu
from jax.experimental.pallas import tpu_sc as plsc

N1 = 5000
N2 = 5000
D_IN = 128
D_HID = 64
DW = 2 * D_HID
E = 320000

NC = 2
NS = 16
NW = NC * NS
EPW = E // NW
CS = 80
NCHS = EPW // CS
CP = 40
NCHP = EPW // CP
NBUF = 5
NPAD = 5120
RPS = NPAD // NS
RBLK = RPS // CS

F32 = jnp.float32
I32 = jnp.int32


def _dot(a, b):
    return lax.dot_general(a, b, (((1,), (0,)), ((), ())),
                           precision=lax.Precision.HIGHEST,
                           preferred_element_type=F32)


def _pre_body(x1, x2, w1e, w2e, w1a, w2a, hcat, p1, p2):
    hcat[...] = jnp.concatenate(
        [_dot(x1[...], w1e[...]), _dot(x2[...], w2e[...])], axis=1)
    p1[...] = _dot(x1[...], w1a[...])
    p2[...] = _dot(x2[...], w2a[...])


_pre = pl.pallas_call(
    _pre_body,
    out_shape=[jax.ShapeDtypeStruct((N1, DW), F32),
               jax.ShapeDtypeStruct((N1, D_HID), F32),
               jax.ShapeDtypeStruct((N2, D_HID), F32)],
)


def _mid_body(hcat, p1, p2, a1, a2, w1b, w2b, gcat):
    e1 = jnp.maximum(hcat[:, :D_HID] + a1[0, :N1, D_HID:] + a1[1, :N1, D_HID:],
                     0.0)
    e2 = jnp.maximum(hcat[:, D_HID:] + a2[0, :N2, :D_HID] + a2[1, :N2, :D_HID],
                     0.0)
    g1 = p1[...] + _dot(e1, w1b[...])
    g2 = p2[...] + _dot(e2, w2b[...])
    gcat[...] = jnp.concatenate([g1, g2], axis=1)


_mid = pl.pallas_call(
    _mid_body,
    out_shape=jax.ShapeDtypeStruct((N1, DW), F32),
)


def _post_body(gcat, b1, b2, ecat):
    em = jnp.maximum(gcat[:, :D_HID] + b1[0, :N1, D_HID:] + b1[1, :N1, D_HID:],
                     0.0)
    ed = jnp.maximum(gcat[:, D_HID:] + b2[0, :N2, :D_HID] + b2[1, :N2, :D_HID],
                     0.0)
    ecat[...] = jnp.concatenate([em, ed], axis=1)


_post = pl.pallas_call(
    _post_body,
    out_shape=jax.ShapeDtypeStruct((N1, DW), F32),
)

_RB = 4000


def _dotred_body(a, m, y):
    y[...] = _dot(a[...], m[...])


_NR = E * 16 // 128

_dotred = pl.pallas_call(
    _dotred_body,
    grid=(_NR // _RB,),
    in_specs=[
        pl.BlockSpec((_RB, 128), lambda i: (i, 0)),
        pl.BlockSpec((128, 8), lambda i: (0, 0)),
    ],
    out_specs=pl.BlockSpec((_RB, 8), lambda i: (i, 0)),
    out_shape=jax.ShapeDtypeStruct((_NR, 8), F32),
)


_MESH = plsc.VectorSubcoreMesh(core_axis_name="c", subcore_axis_name="s")


def _stage_idx(slab, c, dst_buf):
    for k in range(CS // 16):
        dst_buf[pl.ds(k * 16, 16)] = slab[pl.ds(c * CS + k * 16, 16)]


@functools.partial(
    pl.kernel,
    out_type=[jax.ShapeDtypeStruct((NC, NPAD, DW), F32)] * 2,
    mesh=_MESH,
    scratch_types=[
        pltpu.VMEM((EPW,), I32),
        pltpu.VMEM((EPW,), I32),
        pltpu.VMEM_SHARED((NPAD, DW), F32),
        [pltpu.VMEM((CS,), I32) for _ in range(NBUF)],
        [pltpu.VMEM((CS, DW), F32) for _ in range(NBUF)],
        [pltpu.SemaphoreType.DMA for _ in range(NBUF)],
        [pltpu.SemaphoreType.DMA for _ in range(NBUF)],
    ],
)
def _segsum(hcat, src, dst, o1, o2, src_v, dst_v, acc,
            scidx, rows, sems, scs):
    cid = lax.axis_index("c")
    sid = lax.axis_index("s")
    wid = sid * NC + cid

    pltpu.sync_copy(src.at[wid], src_v)
    pltpu.sync_copy(dst.at[wid], dst_v)

    def run_direction(gslab, sslab, out):

        @pl.loop(0, CS)
        def _z(r):
            for k in range(DW // 16):
                rows[0][r, pl.ds(k * 16, 16)] = jnp.zeros((16,), F32)

        for j in range(RBLK):
            blk = pl.ds(sid * RPS + j * CS, CS)
            pltpu.sync_copy(rows[0], acc.at[blk])
        plsc.subcore_barrier()

        def gsrc(c, b):
            return hcat.at[gslab.at[pl.ds(c * CS, CS)]]

        def prefetch(c, b):
            _stage_idx(sslab, c, scidx[b])
            pltpu.async_copy(gsrc(c, b), rows[b], sems[b])

        for b in range(NBUF):
            prefetch(b, b)

        @pl.loop(0, NCHS, step=NBUF)
        def _chunk(c0):
            for b in range(NBUF):
                pltpu.make_async_copy(gsrc(c0 + b, b), rows[b], sems[b]).wait()
                pltpu.async_copy(rows[b], acc.at[scidx[b]], scs[b],
                                 add=True)
            for b in range(NBUF):
                c = c0 + b

                @pl.when(c + NBUF < NCHS)
                def _():
                    pltpu.make_async_copy(rows[b], acc.at[scidx[b]],
                                          scs[b]).wait()
                    prefetch(c + NBUF, b)

        for b in range(NBUF):
            pltpu.make_async_copy(rows[b], acc.at[scidx[b]], scs[b]).wait()
        plsc.subcore_barrier()
        for j in range(RBLK):
            blk = pl.ds(sid * RPS + j * CS, CS)
            pltpu.sync_copy(acc.at[blk], rows[0])
            pltpu.sync_copy(rows[0], out.at[cid, blk])
        plsc.subcore_barrier()

    run_direction(dst_v, src_v, o1)
    run_direction(src_v, dst_v, o2)


@functools.partial(
    pl.kernel,
    out_type=jax.ShapeDtypeStruct((E * 16,), F32),
    mesh=_MESH,
    scratch_types=[
        pltpu.VMEM((2 * EPW,), I32),
        [pltpu.VMEM((2 * CP, DW), F32) for _ in range(NBUF)],
        [pltpu.VMEM((CP * 16,), F32) for _ in range(NBUF)],
        [pltpu.SemaphoreType.DMA for _ in range(NBUF)],
        [pltpu.SemaphoreType.DMA for _ in range(NBUF)],
    ],
)
def _pairgather(ecat, cidx, oy, c_v, rows, pbuf, sg, swy):
    cid = lax.axis_index("c")
    sid = lax.axis_index("s")
    wid = sid * NC + cid
    base = wid * EPW

    pltpu.sync_copy(cidx.at[wid], c_v)

    def gsrc(c):
        return ecat.at[c_v.at[pl.ds(c * 2 * CP, 2 * CP)]]

    def prefetch(c, b):
        pltpu.async_copy(gsrc(c), rows[b], sg[b])

    for b in range(NBUF):
        prefetch(b, b)

    @pl.loop(0, NCHP, step=NBUF)
    def _chunk(c0):
        for b in range(NBUF):
            c = c0 + b
            pltpu.make_async_copy(gsrc(c), rows[b], sg[b]).wait()

            out_span = pl.ds((base + c * CP) * 16, CP * 16)

            @pl.when(c >= NBUF)
            def _():
                pltpu.make_async_copy(pbuf[b], oy.at[out_span], swy[b]).wait()

            for e in range(CP):
                v = None
                for k in range(D_HID // 16):
                    mk = rows[b][e, pl.ds(16 * k, 16)]
                    dk = rows[b][CP + e, pl.ds(D_HID + 16 * k, 16)]
                    v = mk * dk if v is None else v + mk * dk
                pbuf[b][pl.ds(e * 16, 16)] = v

            pltpu.async_copy(pbuf[b], oy.at[out_span], swy[b])

            @pl.when(c + NBUF < NCHP)
            def _():
                prefetch(c + NBUF, b)

    for b in range(NBUF):
        c = NCHP - NBUF + b
        out_span = pl.ds((base + c * CP) * 16, CP * 16)
        pltpu.make_async_copy(pbuf[b], oy.at[out_span], swy[b]).wait()


def kernel(x1, x2, edge_index_mp, edge_index, W1_enc, W2_enc, W1_out, W2_out):
    src = edge_index_mp[0].astype(I32).reshape(NW, EPW)
    dst = edge_index_mp[1].astype(I32).reshape(NW, EPW)
    ci = jnp.stack([edge_index[0].astype(I32).reshape(NW, NCHP, CP),
                    edge_index[1].astype(I32).reshape(NW, NCHP, CP)],
                   axis=2).reshape(NW, 2 * EPW)
    w1a, w1b = W1_out[:D_IN], W1_out[D_IN:]
    w2a, w2b = W2_out[:D_IN], W2_out[D_IN:]
    mask8 = jnp.kron(jnp.eye(8, dtype=F32), jnp.ones((16, 1), F32))

    hcat, p1, p2 = _pre(x1, x2, W1_enc, W2_enc, w1a, w2a)
    a1, a2 = _segsum(hcat, src, dst)
    gcat = _mid(hcat, p1, p2, a1, a2, w1b, w2b)
    b1, b2 = _segsum(gcat, src, dst)
    ecat = _post(gcat, b1, b2)
    y16 = _pairgather(ecat, ci)
    y = _dotred(y16.reshape(_NR, 128), mask8)
    return y.reshape(E, 1)


# speedup vs baseline: 1.0844x; 1.0844x over previous
import functools

import jax
import jax.numpy as jnp
from jax import lax
from jax.experimental import pallas as pl
from jax.experimental.pallas import tpu as pltpu
from jax.experimental.pallas import tpu_sc as plsc

N1 = 5000
N2 = 5000
D_IN = 128
D_HID = 64
DW = 2 * D_HID
E = 320000

NC = 2
NS = 16
NW = NC * NS
EPW = E // NW
CS = 80
NCHS = EPW // CS
CP = 40
NCHP = EPW // CP
NBUF = 5
NPAD = 5120
RPS = NPAD // NS
RBLK = RPS // CS

F32 = jnp.float32
I32 = jnp.int32


def _dot(a, b):
    return lax.dot_general(a, b, (((1,), (0,)), ((), ())),
                           precision=lax.Precision.HIGHEST,
                           preferred_element_type=F32)


def _pre_body(x1, x2, w1e, w2e, w1a, w2a, hcat, p1, p2):
    hcat[...] = jnp.concatenate(
        [_dot(x1[...], w1e[...]), _dot(x2[...], w2e[...])], axis=1)
    p1[...] = _dot(x1[...], w1a[...])
    p2[...] = _dot(x2[...], w2a[...])


_pre = pl.pallas_call(
    _pre_body,
    out_shape=[jax.ShapeDtypeStruct((N1, DW), F32),
               jax.ShapeDtypeStruct((N1, D_HID), F32),
               jax.ShapeDtypeStruct((N2, D_HID), F32)],
)


def _mid_body(hcat, p1, p2, a1, a2, w1b, w2b, gcat):
    e1 = jnp.maximum(hcat[:, :D_HID] + a1[0, :N1, D_HID:] + a1[1, :N1, D_HID:],
                     0.0)
    e2 = jnp.maximum(hcat[:, D_HID:] + a2[0, :N2, :D_HID] + a2[1, :N2, :D_HID],
                     0.0)
    g1 = p1[...] + _dot(e1, w1b[...])
    g2 = p2[...] + _dot(e2, w2b[...])
    gcat[...] = jnp.concatenate([g1, g2], axis=1)


_mid = pl.pallas_call(
    _mid_body,
    out_shape=jax.ShapeDtypeStruct((N1, DW), F32),
)


def _post_body(gcat, b1, b2, ecat):
    em = jnp.maximum(gcat[:, :D_HID] + b1[0, :N1, D_HID:] + b1[1, :N1, D_HID:],
                     0.0)
    ed = jnp.maximum(gcat[:, D_HID:] + b2[0, :N2, :D_HID] + b2[1, :N2, :D_HID],
                     0.0)
    ecat[...] = jnp.concatenate([em, ed], axis=1)


_post = pl.pallas_call(
    _post_body,
    out_shape=jax.ShapeDtypeStruct((N1, DW), F32),
)

_RB = 4000


def _dotred_body(a, m, y):
    y[...] = _dot(a[...], m[...])


_NR = E * 16 // 128

_dotred = pl.pallas_call(
    _dotred_body,
    grid=(_NR // _RB,),
    in_specs=[
        pl.BlockSpec((_RB, 128), lambda i: (i, 0)),
        pl.BlockSpec((128, 8), lambda i: (0, 0)),
    ],
    out_specs=pl.BlockSpec((_RB, 8), lambda i: (i, 0)),
    out_shape=jax.ShapeDtypeStruct((_NR, 8), F32),
)


_MESH = plsc.VectorSubcoreMesh(core_axis_name="c", subcore_axis_name="s")


def _stage_idx(slab, c, dst_buf):
    for k in range(CS // 16):
        dst_buf[pl.ds(k * 16, 16)] = slab[pl.ds(c * CS + k * 16, 16)]


@functools.partial(
    pl.kernel,
    out_type=[jax.ShapeDtypeStruct((NC, NPAD, DW), F32)] * 2,
    mesh=_MESH,
    scratch_types=[
        pltpu.VMEM((EPW,), I32),
        pltpu.VMEM((EPW,), I32),
        pltpu.VMEM_SHARED((NPAD, DW), F32),
        [pltpu.VMEM((CS,), I32) for _ in range(NBUF)],
        [pltpu.VMEM((CS, DW), F32) for _ in range(NBUF)],
        [pltpu.SemaphoreType.DMA for _ in range(NBUF)],
    ],
)
def _segsum(hcat, src, dst, o1, o2, src_v, dst_v, acc,
            scidx, rows, sems):
    cid = lax.axis_index("c")
    sid = lax.axis_index("s")
    wid = sid * NC + cid

    pltpu.sync_copy(src.at[wid], src_v)
    pltpu.sync_copy(dst.at[wid], dst_v)

    def run_direction(gslab, sslab, out):

        @pl.loop(0, CS)
        def _z(r):
            for k in range(DW // 16):
                rows[0][r, pl.ds(k * 16, 16)] = jnp.zeros((16,), F32)

        for j in range(RBLK):
            blk = pl.ds(sid * RPS + j * CS, CS)
            pltpu.sync_copy(rows[0], acc.at[blk])
        plsc.subcore_barrier()

        def gsrc(c, b):
            return hcat.at[gslab.at[pl.ds(c * CS, CS)]]

        def prefetch(c, b):
            _stage_idx(sslab, c, scidx[b])
            pltpu.async_copy(gsrc(c, b), rows[b], sems[b])

        for b in range(NBUF):
            prefetch(b, b)

        @pl.loop(0, NCHS, step=NBUF)
        def _chunk(c0):
            for b in range(NBUF):
                c = c0 + b
                pltpu.make_async_copy(gsrc(c, b), rows[b], sems[b]).wait()
                pltpu.sync_copy(rows[b], acc.at[scidx[b]], add=True)

                @pl.when(c + NBUF < NCHS)
                def _():
                    prefetch(c + NBUF, b)

        plsc.subcore_barrier()
        for j in range(RBLK):
            blk = pl.ds(sid * RPS + j * CS, CS)
            pltpu.sync_copy(acc.at[blk], rows[0])
            pltpu.sync_copy(rows[0], out.at[cid, blk])
        plsc.subcore_barrier()

    run_direction(dst_v, src_v, o1)
    run_direction(src_v, dst_v, o2)


@functools.partial(
    pl.kernel,
    out_type=jax.ShapeDtypeStruct((E * 16,), F32),
    mesh=_MESH,
    scratch_types=[
        pltpu.VMEM((2 * EPW,), I32),
        [pltpu.VMEM((2 * CP, DW), F32) for _ in range(NBUF)],
        [pltpu.VMEM((CP * 16,), F32) for _ in range(NBUF)],
        [pltpu.SemaphoreType.DMA for _ in range(NBUF)],
        [pltpu.SemaphoreType.DMA for _ in range(NBUF)],
    ],
)
def _pairgather(ecat, cidx, oy, c_v, rows, pbuf, sg, swy):
    cid = lax.axis_index("c")
    sid = lax.axis_index("s")
    wid = sid * NC + cid
    base = wid * EPW

    pltpu.sync_copy(cidx.at[wid], c_v)

    def gsrc(c):
        return ecat.at[c_v.at[pl.ds(c * 2 * CP, 2 * CP)]]

    def prefetch(c, b):
        pltpu.async_copy(gsrc(c), rows[b], sg[b])

    for b in range(NBUF):
        prefetch(b, b)

    @pl.loop(0, NCHP, step=NBUF)
    def _chunk(c0):
        for b in range(NBUF):
            c = c0 + b
            pltpu.make_async_copy(gsrc(c), rows[b], sg[b]).wait()

            out_span = pl.ds((base + c * CP) * 16, CP * 16)

            @pl.when(c >= NBUF)
            def _():
                pltpu.make_async_copy(pbuf[b], oy.at[out_span], swy[b]).wait()

            for e in range(CP):
                v = None
                for k in range(D_HID // 16):
                    mk = rows[b][e, pl.ds(16 * k, 16)]
                    dk = rows[b][CP + e, pl.ds(D_HID + 16 * k, 16)]
                    v = mk * dk if v is None else v + mk * dk
                pbuf[b][pl.ds(e * 16, 16)] = v

            pltpu.async_copy(pbuf[b], oy.at[out_span], swy[b])

            @pl.when(c + NBUF < NCHP)
            def _():
                prefetch(c + NBUF, b)

    for b in range(NBUF):
        c = NCHP - NBUF + b
        out_span = pl.ds((base + c * CP) * 16, CP * 16)
        pltpu.make_async_copy(pbuf[b], oy.at[out_span], swy[b]).wait()


def kernel(x1, x2, edge_index_mp, edge_index, W1_enc, W2_enc, W1_out, W2_out):
    src = edge_index_mp[0].astype(I32).reshape(NW, EPW)
    dst = edge_index_mp[1].astype(I32).reshape(NW, EPW)
    ci = jnp.stack([edge_index[0].astype(I32).reshape(NW, NCHP, CP),
                    edge_index[1].astype(I32).reshape(NW, NCHP, CP)],
                   axis=2).reshape(NW, 2 * EPW)
    w1a, w1b = W1_out[:D_IN], W1_out[D_IN:]
    w2a, w2b = W2_out[:D_IN], W2_out[D_IN:]
    mask8 = jnp.kron(jnp.eye(8, dtype=F32), jnp.ones((16, 1), F32))

    hcat, p1, p2 = _pre(x1, x2, W1_enc, W2_enc, w1a, w2a)
    a1, a2 = _segsum(hcat, src, dst)
    gcat = _mid(hcat, p1, p2, a1, a2, w1b, w2b)
    b1, b2 = _segsum(gcat, src, dst)
    ecat = _post(gcat, b1, b2)
    y16 = _pairgather(ecat, ci)
    y = _dotred(y16.reshape(_NR, 128), mask8)
    return y.reshape(E, 1)


# speedup vs baseline: 1.3571x; 1.2515x over previous
import functools

import jax
import jax.numpy as jnp
from jax import lax
from jax.experimental import pallas as pl
from jax.experimental.pallas import tpu as pltpu
from jax.experimental.pallas import tpu_sc as plsc

N1 = 5000
N2 = 5000
D_IN = 128
D_HID = 64
DW = 2 * D_HID
E = 320000

NC = 2
NS = 16
NW = NC * NS
EPW = E // NW
CS = 80
NCHS = EPW // CS
CP = 40
NCHP = EPW // CP
NBUF = 5
NPAD = 5120
RPS = NPAD // NS
RBLK = RPS // CS

F32 = jnp.float32
I32 = jnp.int32


def _dot(a, b):
    return lax.dot_general(a, b, (((1,), (0,)), ((), ())),
                           precision=lax.Precision.HIGHEST,
                           preferred_element_type=F32)


def _pre_body(x1, x2, w1e, w2e, w1a, w2a, hcat, p1, p2):
    hcat[...] = jnp.concatenate(
        [_dot(x1[...], w1e[...]), _dot(x2[...], w2e[...])], axis=1)
    p1[...] = _dot(x1[...], w1a[...])
    p2[...] = _dot(x2[...], w2a[...])


_pre = pl.pallas_call(
    _pre_body,
    out_shape=[jax.ShapeDtypeStruct((N1, DW), F32),
               jax.ShapeDtypeStruct((N1, D_HID), F32),
               jax.ShapeDtypeStruct((N2, D_HID), F32)],
)


def _mid_body(hcat, p1, p2, a1, a2, w1b, w2b, gcat):
    e1 = jnp.maximum(hcat[:, :D_HID] + a1[0, :N1, D_HID:] + a1[1, :N1, D_HID:],
                     0.0)
    e2 = jnp.maximum(hcat[:, D_HID:] + a2[0, :N2, :D_HID] + a2[1, :N2, :D_HID],
                     0.0)
    g1 = p1[...] + _dot(e1, w1b[...])
    g2 = p2[...] + _dot(e2, w2b[...])
    gcat[...] = jnp.concatenate([g1, g2], axis=1)


_mid = pl.pallas_call(
    _mid_body,
    out_shape=jax.ShapeDtypeStruct((N1, DW), F32),
)


def _post_body(gcat, b1, b2, ecat):
    em = jnp.maximum(gcat[:, :D_HID] + b1[0, :N1, D_HID:] + b1[1, :N1, D_HID:],
                     0.0)
    ed = jnp.maximum(gcat[:, D_HID:] + b2[0, :N2, :D_HID] + b2[1, :N2, :D_HID],
                     0.0)
    cat = jnp.concatenate([em, ed], axis=1)
    ecat[...] = jnp.concatenate([cat, jnp.zeros((NDP - N2, DW), F32)], axis=0)


NDT = 40
NDP = NDT * 128

_post = pl.pallas_call(
    _post_body,
    out_shape=jax.ShapeDtypeStruct((NDP, DW), F32),
)

_BM = 1000


_BD = 1024


def _score_body(a, b, y):
    y[...] = lax.dot_general(
        a[:, :D_HID], b[:, D_HID:], (((1,), (1,)), ((), ())),
        precision=lax.Precision.HIGHEST,
        preferred_element_type=F32).reshape(_BM, _BD // 128, 128)


_score = pl.pallas_call(
    _score_body,
    grid=(N1 // _BM, NDP // _BD),
    in_specs=[
        pl.BlockSpec((_BM, DW), lambda i, j: (i, 0)),
        pl.BlockSpec((_BD, DW), lambda i, j: (j, 0)),
    ],
    out_specs=pl.BlockSpec((_BM, _BD // 128, 128), lambda i, j: (i, j, 0)),
    out_shape=jax.ShapeDtypeStruct((N1, NDT, 128), F32),
)


_MESH = plsc.VectorSubcoreMesh(core_axis_name="c", subcore_axis_name="s")


def _stage_idx(slab, c, dst_buf):
    for k in range(CS // 16):
        dst_buf[pl.ds(k * 16, 16)] = slab[pl.ds(c * CS + k * 16, 16)]


@functools.partial(
    pl.kernel,
    out_type=[jax.ShapeDtypeStruct((NC, NPAD, DW), F32)] * 2,
    mesh=_MESH,
    scratch_types=[
        pltpu.VMEM((EPW,), I32),
        pltpu.VMEM((EPW,), I32),
        pltpu.VMEM_SHARED((NPAD, DW), F32),
        [pltpu.VMEM((CS,), I32) for _ in range(NBUF)],
        [pltpu.VMEM((CS, DW), F32) for _ in range(NBUF)],
        [pltpu.SemaphoreType.DMA for _ in range(NBUF)],
    ],
)
def _segsum(hcat, src, dst, o1, o2, src_v, dst_v, acc,
            scidx, rows, sems):
    cid = lax.axis_index("c")
    sid = lax.axis_index("s")
    wid = sid * NC + cid

    pltpu.sync_copy(src.at[wid], src_v)
    pltpu.sync_copy(dst.at[wid], dst_v)

    def run_direction(gslab, sslab, out):

        @pl.loop(0, CS)
        def _z(r):
            for k in range(DW // 16):
                rows[0][r, pl.ds(k * 16, 16)] = jnp.zeros((16,), F32)

        for j in range(RBLK):
            blk = pl.ds(sid * RPS + j * CS, CS)
            pltpu.sync_copy(rows[0], acc.at[blk])
        plsc.subcore_barrier()

        def gsrc(c, b):
            return hcat.at[gslab.at[pl.ds(c * CS, CS)]]

        def prefetch(c, b):
            _stage_idx(sslab, c, scidx[b])
            pltpu.async_copy(gsrc(c, b), rows[b], sems[b])

        for b in range(NBUF):
            prefetch(b, b)

        @pl.loop(0, NCHS, step=NBUF)
        def _chunk(c0):
            for b in range(NBUF):
                c = c0 + b
                pltpu.make_async_copy(gsrc(c, b), rows[b], sems[b]).wait()
                pltpu.sync_copy(rows[b], acc.at[scidx[b]], add=True)

                @pl.when(c + NBUF < NCHS)
                def _():
                    prefetch(c + NBUF, b)

        plsc.subcore_barrier()
        for j in range(RBLK):
            blk = pl.ds(sid * RPS + j * CS, CS)
            pltpu.sync_copy(acc.at[blk], rows[0])
            pltpu.sync_copy(rows[0], out.at[cid, blk])
        plsc.subcore_barrier()

    run_direction(dst_v, src_v, o1)
    run_direction(src_v, dst_v, o2)


@functools.partial(
    pl.kernel,
    out_type=jax.ShapeDtypeStruct((E,), F32),
    mesh=_MESH,
    scratch_types=[
        pltpu.VMEM((EPW,), I32),
        pltpu.VMEM((EPW,), F32),
    ],
)
def _qgather(yflat, qidx, oy, idx_v, val_v):
    cid = lax.axis_index("c")
    sid = lax.axis_index("s")
    wid = sid * NC + cid

    pltpu.sync_copy(qidx.at[wid], idx_v)
    pltpu.sync_copy(yflat.at[idx_v], val_v)
    pltpu.sync_copy(val_v, oy.at[pl.ds(wid * EPW, EPW)])


def kernel(x1, x2, edge_index_mp, edge_index, W1_enc, W2_enc, W1_out, W2_out):
    src = edge_index_mp[0].astype(I32).reshape(NW, EPW)
    dst = edge_index_mp[1].astype(I32).reshape(NW, EPW)
    qi = (edge_index[0].astype(I32) * NDP
          + edge_index[1].astype(I32)).reshape(NW, EPW)
    w1a, w1b = W1_out[:D_IN], W1_out[D_IN:]
    w2a, w2b = W2_out[:D_IN], W2_out[D_IN:]

    hcat, p1, p2 = _pre(x1, x2, W1_enc, W2_enc, w1a, w2a)
    a1, a2 = _segsum(hcat, src, dst)
    gcat = _mid(hcat, p1, p2, a1, a2, w1b, w2b)
    b1, b2 = _segsum(gcat, src, dst)
    ecat = _post(gcat, b1, b2)
    yall = _score(ecat, ecat)
    y = _qgather(yall.reshape(N1 * NDP), qi)
    return y.reshape(E, 1)


# speedup vs baseline: 1.4718x; 1.0845x over previous
import functools

import jax
import jax.numpy as jnp
from jax import lax
from jax.experimental import pallas as pl
from jax.experimental.pallas import tpu as pltpu
from jax.experimental.pallas import tpu_sc as plsc

N1 = 5000
N2 = 5000
D_IN = 128
D_HID = 64
DW = 2 * D_HID
E = 320000

NC = 2
NS = 16
NW = NC * NS
EPW = E // NW
CS = 80
NCHS = EPW // CS
CP = 40
NCHP = EPW // CP
NBUF = 5
NPAD = 5120
RPS = NPAD // NS
RBLK = RPS // CS

F32 = jnp.float32
I32 = jnp.int32


def _dot(a, b):
    return lax.dot_general(a, b, (((1,), (0,)), ((), ())),
                           precision=lax.Precision.HIGHEST,
                           preferred_element_type=F32)


def _pre_body(x1, x2, w1e, w2e, w1a, w2a, hcat, p1, p2):
    hcat[...] = jnp.concatenate(
        [_dot(x1[...], w1e[...]), _dot(x2[...], w2e[...])], axis=1)
    p1[...] = _dot(x1[...], w1a[...])
    p2[...] = _dot(x2[...], w2a[...])


_pre = pl.pallas_call(
    _pre_body,
    out_shape=[jax.ShapeDtypeStruct((N1, DW), F32),
               jax.ShapeDtypeStruct((N1, D_HID), F32),
               jax.ShapeDtypeStruct((N2, D_HID), F32)],
)


def _mid_body(hcat, p1, p2, a1, a2, w1b, w2b, gcat):
    e1 = jnp.maximum(hcat[:, :D_HID] + a1[0, :N1, D_HID:] + a1[1, :N1, D_HID:],
                     0.0)
    e2 = jnp.maximum(hcat[:, D_HID:] + a2[0, :N2, :D_HID] + a2[1, :N2, :D_HID],
                     0.0)
    g1 = p1[...] + _dot(e1, w1b[...])
    g2 = p2[...] + _dot(e2, w2b[...])
    gcat[...] = jnp.concatenate([g1, g2], axis=1)


_mid = pl.pallas_call(
    _mid_body,
    out_shape=jax.ShapeDtypeStruct((N1, DW), F32),
)


def _post_body(gcat, b1, b2, ecat):
    em = jnp.maximum(gcat[:, :D_HID] + b1[0, :N1, D_HID:] + b1[1, :N1, D_HID:],
                     0.0)
    ed = jnp.maximum(gcat[:, D_HID:] + b2[0, :N2, :D_HID] + b2[1, :N2, :D_HID],
                     0.0)
    cat = jnp.concatenate([em, ed], axis=1)
    ecat[...] = jnp.concatenate([cat, jnp.zeros((NDP - N2, DW), F32)], axis=0)


NDT = 40
NDP = NDT * 128

_post = pl.pallas_call(
    _post_body,
    out_shape=jax.ShapeDtypeStruct((NDP, DW), F32),
)

_BM = 1000


_BD = 1024


def _score_body(a, b, y):
    y[...] = lax.dot_general(
        a[:, :D_HID].astype(jnp.bfloat16), b[:, D_HID:].astype(jnp.bfloat16),
        (((1,), (1,)), ((), ())),
        preferred_element_type=F32).reshape(_BM, _BD // 128, 128)


_score = pl.pallas_call(
    _score_body,
    grid=(N1 // _BM, NDP // _BD),
    in_specs=[
        pl.BlockSpec((_BM, DW), lambda i, j: (i, 0)),
        pl.BlockSpec((_BD, DW), lambda i, j: (j, 0)),
    ],
    out_specs=pl.BlockSpec((_BM, _BD // 128, 128), lambda i, j: (i, j, 0)),
    out_shape=jax.ShapeDtypeStruct((N1, NDT, 128), F32),
)


_MESH = plsc.VectorSubcoreMesh(core_axis_name="c", subcore_axis_name="s")


def _stage_idx(slab, c, dst_buf):
    for k in range(CS // 16):
        dst_buf[pl.ds(k * 16, 16)] = slab[pl.ds(c * CS + k * 16, 16)]


@functools.partial(
    pl.kernel,
    out_type=[jax.ShapeDtypeStruct((NC, NPAD, DW), F32)] * 2,
    mesh=_MESH,
    scratch_types=[
        pltpu.VMEM((EPW,), I32),
        pltpu.VMEM((EPW,), I32),
        pltpu.VMEM_SHARED((NPAD, DW), F32),
        [pltpu.VMEM((CS,), I32) for _ in range(NBUF)],
        [pltpu.VMEM((CS, DW), F32) for _ in range(NBUF)],
        [pltpu.SemaphoreType.DMA for _ in range(NBUF)],
    ],
)
def _segsum(hcat, src, dst, o1, o2, src_v, dst_v, acc,
            scidx, rows, sems):
    cid = lax.axis_index("c")
    sid = lax.axis_index("s")
    wid = sid * NC + cid

    pltpu.sync_copy(src.at[wid], src_v)
    pltpu.sync_copy(dst.at[wid], dst_v)

    def run_direction(gslab, sslab, out):

        @pl.loop(0, CS)
        def _z(r):
            for k in range(DW // 16):
                rows[0][r, pl.ds(k * 16, 16)] = jnp.zeros((16,), F32)

        for j in range(RBLK):
            blk = pl.ds(sid * RPS + j * CS, CS)
            pltpu.sync_copy(rows[0], acc.at[blk])
        plsc.subcore_barrier()

        def gsrc(c, b):
            return hcat.at[gslab.at[pl.ds(c * CS, CS)]]

        def prefetch(c, b):
            _stage_idx(sslab, c, scidx[b])
            pltpu.async_copy(gsrc(c, b), rows[b], sems[b])

        for b in range(NBUF):
            prefetch(b, b)

        @pl.loop(0, NCHS, step=NBUF)
        def _chunk(c0):
            for b in range(NBUF):
                c = c0 + b
                pltpu.make_async_copy(gsrc(c, b), rows[b], sems[b]).wait()
                pltpu.sync_copy(rows[b], acc.at[scidx[b]], add=True)

                @pl.when(c + NBUF < NCHS)
                def _():
                    prefetch(c + NBUF, b)

        plsc.subcore_barrier()
        for j in range(RBLK):
            blk = pl.ds(sid * RPS + j * CS, CS)
            pltpu.sync_copy(acc.at[blk], rows[0])
            pltpu.sync_copy(rows[0], out.at[cid, blk])
        plsc.subcore_barrier()

    run_direction(dst_v, src_v, o1)
    run_direction(src_v, dst_v, o2)


@functools.partial(
    pl.kernel,
    out_type=jax.ShapeDtypeStruct((E,), F32),
    mesh=_MESH,
    scratch_types=[
        pltpu.VMEM((EPW,), I32),
        pltpu.VMEM((EPW,), F32),
    ],
)
def _qgather(yflat, qidx, oy, idx_v, val_v):
    cid = lax.axis_index("c")
    sid = lax.axis_index("s")
    wid = sid * NC + cid

    pltpu.sync_copy(qidx.at[wid], idx_v)
    pltpu.sync_copy(yflat.at[idx_v], val_v)
    pltpu.sync_copy(val_v, oy.at[pl.ds(wid * EPW, EPW)])


def kernel(x1, x2, edge_index_mp, edge_index, W1_enc, W2_enc, W1_out, W2_out):
    src = edge_index_mp[0].astype(I32).reshape(NW, EPW)
    dst = edge_index_mp[1].astype(I32).reshape(NW, EPW)
    qi = (edge_index[0].astype(I32) * NDP
          + edge_index[1].astype(I32)).reshape(NW, EPW)
    w1a, w1b = W1_out[:D_IN], W1_out[D_IN:]
    w2a, w2b = W2_out[:D_IN], W2_out[D_IN:]

    hcat, p1, p2 = _pre(x1, x2, W1_enc, W2_enc, w1a, w2a)
    a1, a2 = _segsum(hcat, src, dst)
    gcat = _mid(hcat, p1, p2, a1, a2, w1b, w2b)
    b1, b2 = _segsum(gcat, src, dst)
    ecat = _post(gcat, b1, b2)
    yall = _score(ecat, ecat)
    y = _qgather(yall.reshape(N1 * NDP), qi)
    return y.reshape(E, 1)
